# Initial kernel scaffold; baseline (speedup 1.0000x reference)
import jax
import jax.numpy as jnp
from jax.experimental import pallas as pl


def kernel(x, edge_index, edge_type, emb, Ws1, bs1, Ws2, bs2, Wt1, bt1, Wt2, bt2):
    raise NotImplementedError("write your pallas kernel here")


# speedup vs baseline: 1.7028x; 1.7028x over previous
import functools

import jax
import jax.numpy as jnp
from jax import lax
from jax.experimental import pallas as pl
from jax.experimental.pallas import tpu as pltpu
from jax.experimental.pallas import tpu_sc as plsc

N = 10000
E = 320000
IN = 128
HID = 256
HALF = 64
NT = 16
NSUB = 16
NCORE = 2
CH = 80
EPT = E // NSUB
NCHUNK = EPT // CH
RPT = N // NSUB


def _tc_pre_body(x_ref, emb_ref, wt1_ref, bt1_ref, ws1_ref, bs1_ref,
                 ws2_ref, bs2_ref, y_ref, t_ref, sf_ref):
    x = x_ref[...]
    wt1 = wt1_ref[...]
    y_ref[...] = jnp.dot(x, wt1[IN:, :], preferred_element_type=jnp.float32)
    t_ref[...] = (jnp.dot(emb_ref[...], wt1[:IN, :],
                          preferred_element_type=jnp.float32) + bt1_ref[...])
    h = jnp.maximum(jnp.dot(x, ws1_ref[...],
                            preferred_element_type=jnp.float32)
                    + bs1_ref[...], 0.0)
    sf_ref[...] = (jnp.dot(h, ws2_ref[...],
                           preferred_element_type=jnp.float32) + bs2_ref[...])


def _tc_pre(x, emb, Wt1, bt1, Ws1, bs1, Ws2, bs2):
    return pl.pallas_call(
        _tc_pre_body,
        out_shape=[
            jax.ShapeDtypeStruct((N, HID), jnp.float32),
            jax.ShapeDtypeStruct((NT, HID), jnp.float32),
            jax.ShapeDtypeStruct((N, HALF), jnp.float32),
        ],
    )(x, emb, Wt1, bt1.reshape(1, HID), Ws1, bs1.reshape(1, IN),
      Ws2, bs2.reshape(1, HALF))


def _sc_edges_body(src_h, dst_h, et_h, y0_h, y1_h, t0_h, t1_h,
                   h0_out, h1_out, d_out,
                   h_sh, t_v, src_v, dst_v, et_v, rows_v, deg_v,
                   z_v, sem):
    c = lax.axis_index("c")
    s = lax.axis_index("s")

    zero16 = jnp.zeros((16,), jnp.float32)
    one16 = jnp.full((16,), 1.0, jnp.float32)
    lanes = lax.iota(jnp.int32, 16)

    def z_row(i, carry):
        for j in range(8):
            z_v[i, pl.ds(j * 16, 16)] = zero16
        return carry
    lax.fori_loop(0, 128, z_row, 0)

    def zdeg(i, carry):
        deg_v[pl.ds(i * 16, 16)] = zero16
        return carry
    lax.fori_loop(0, 640, zdeg, 0)

    r0 = s * 624
    for k in range(4):
        pltpu.sync_copy(z_v, h_sh.at[pl.ds(r0 + k * 128, 128)])

    @pl.when(s < 15)
    def _():
        pltpu.sync_copy(z_v.at[pl.ds(0, 112)], h_sh.at[pl.ds(r0 + 512, 112)])

    @pl.when(s == 15)
    def _():
        pltpu.sync_copy(z_v, h_sh.at[pl.ds(r0 + 512, 128)])

    @pl.when(c == 0)
    def _():
        pltpu.sync_copy(t0_h, t_v)

    @pl.when(c == 1)
    def _():
        pltpu.sync_copy(t1_h, t_v)

    plsc.subcore_barrier()

    base_t = s * EPT

    def chunk(i, carry):
        e0 = base_t + i * CH
        pltpu.sync_copy(src_h.at[pl.ds(e0, CH)], src_v)
        pltpu.sync_copy(et_h.at[pl.ds(e0, CH)], et_v)
        pltpu.sync_copy(dst_h.at[pl.ds(e0, CH)], dst_v)

        @pl.when(c == 0)
        def _():
            pltpu.async_copy(y0_h.at[src_v], rows_v, sem).wait()

        @pl.when(c == 1)
        def _():
            pltpu.async_copy(y1_h.at[src_v], rows_v, sem).wait()

        def grp(g, gcarry):
            et16 = et_v[pl.ds(g * 16, 16)]
            dst16 = dst_v[pl.ds(g * 16, 16)]

            @pl.when(c == 1)
            def _():
                for i in range(16):
                    plsc.addupdate_scatter(deg_v, [dst16], one16,
                                           mask=lanes == i)
            for i in range(16):
                te = et16[i]
                e = g * 16 + i
                for j in range(8):
                    sl = pl.ds(j * 16, 16)
                    rows_v[e, sl] = jnp.maximum(
                        rows_v[e, sl] + t_v[te, sl], 0.0)
            return gcarry
        lax.fori_loop(0, CH // 16, grp, 0)

        pltpu.sync_copy(rows_v, h_sh.at[dst_v], add=True)
        return carry

    lax.fori_loop(0, NCHUNK, chunk, 0)
    plsc.subcore_barrier()

    def stripe_out(h_out):
        for k in range(4):
            pltpu.sync_copy(h_sh.at[pl.ds(r0 + k * 128, 128)], z_v)
            pltpu.sync_copy(z_v, h_out.at[pl.ds(r0 + k * 128, 128)])

        @pl.when(s < 15)
        def _():
            pltpu.sync_copy(h_sh.at[pl.ds(r0 + 512, 112)],
                            z_v.at[pl.ds(0, 112)])
            pltpu.sync_copy(z_v.at[pl.ds(0, 112)],
                            h_out.at[pl.ds(r0 + 512, 112)])

        @pl.when(s == 15)
        def _():
            pltpu.sync_copy(h_sh.at[pl.ds(r0 + 512, 128)], z_v)
            pltpu.sync_copy(z_v, h_out.at[pl.ds(r0 + 512, 128)])

    @pl.when(c == 0)
    def _():
        stripe_out(h0_out)

    @pl.when(c == 1)
    def _():
        stripe_out(h1_out)
        pltpu.sync_copy(deg_v, d_out.at[s])


def _sc_edges(src, dst, et, y0, y1, t0, t1):
    mesh = plsc.VectorSubcoreMesh(core_axis_name="c", subcore_axis_name="s")
    f = pl.kernel(
        _sc_edges_body,
        out_type=[
            jax.ShapeDtypeStruct((N, IN), jnp.float32),
            jax.ShapeDtypeStruct((N, IN), jnp.float32),
            jax.ShapeDtypeStruct((NSUB, 10240), jnp.float32),
        ],
        mesh=mesh,
        scratch_types=[
            pltpu.VMEM_SHARED((N, IN), jnp.float32),
            pltpu.VMEM((NT, IN), jnp.float32),
            pltpu.VMEM((CH,), jnp.int32),
            pltpu.VMEM((CH,), jnp.int32),
            pltpu.VMEM((CH,), jnp.int32),
            pltpu.VMEM((CH, IN), jnp.float32),
            pltpu.VMEM((10240,), jnp.float32),
            pltpu.VMEM((128, IN), jnp.float32),
            pltpu.SemaphoreType.DMA,
        ],
        compiler_params=pltpu.CompilerParams(use_tc_tiling_on_sc=False,
                                             needs_layout_passes=False),
    )
    return f(src, dst, et, y0, y1, t0, t1)


def _tc_post_body(h0_ref, h1_ref, d_ref, sf_ref, wt2_ref, bt2_ref, out_ref):
    wt2 = wt2_ref[...]
    bt2 = bt2_ref[...]
    bt2_rep = jnp.broadcast_to(bt2, (NSUB, HALF))
    extra = lax.dot_general(d_ref[...], bt2_rep,
                            (((0,), (0,)), ((), ())),
                            preferred_element_type=jnp.float32)
    msg = (jnp.dot(h0_ref[...], wt2[:IN, :], preferred_element_type=jnp.float32)
           + jnp.dot(h1_ref[...], wt2[IN:, :], preferred_element_type=jnp.float32)
           + extra[:N, :])
    out_ref[...] = jnp.concatenate([sf_ref[...], msg], axis=-1)


def _tc_post(h0, h1, d, sf, Wt2, bt2):
    return pl.pallas_call(
        _tc_post_body,
        out_shape=jax.ShapeDtypeStruct((N, 2 * HALF), jnp.float32),
    )(h0, h1, d, sf, Wt2, bt2.reshape(1, HALF))


def kernel(x, edge_index, edge_type, emb, Ws1, bs1, Ws2, bs2,
           Wt1, bt1, Wt2, bt2):
    src = edge_index[0].astype(jnp.int32)
    dst = edge_index[1].astype(jnp.int32)
    et = edge_type.astype(jnp.int32)

    y, t, sf = _tc_pre(x, emb, Wt1, bt1, Ws1, bs1, Ws2, bs2)
    y0 = y[:, :IN] + 0.0
    y1 = y[:, IN:] + 0.0
    t0 = t[:, :IN] + 0.0
    t1 = t[:, IN:] + 0.0

    h0, h1, d = _sc_edges(src, dst, et, y0, y1, t0, t1)
    return _tc_post(h0, h1, d, sf, Wt2, bt2)


# speedup vs baseline: 2.5550x; 1.5005x over previous
import functools

import jax
import jax.numpy as jnp
from jax import lax
from jax.experimental import pallas as pl
from jax.experimental.pallas import tpu as pltpu
from jax.experimental.pallas import tpu_sc as plsc

N = 10000
E = 320000
IN = 128
HID = 256
HALF = 64
NT = 16
NSUB = 16
NCORE = 2
CH = 80
EPT = E // NSUB
NCHUNK = EPT // CH
RPT = N // NSUB


def _tc_pre_body(x_ref, emb_ref, wt1_ref, bt1_ref, ws1_ref, bs1_ref,
                 ws2_ref, bs2_ref, y_ref, t_ref, sf_ref):
    x = x_ref[...]
    wt1 = wt1_ref[...]
    y_ref[...] = jnp.dot(x, wt1[IN:, :], preferred_element_type=jnp.float32)
    t_ref[...] = (jnp.dot(emb_ref[...], wt1[:IN, :],
                          preferred_element_type=jnp.float32) + bt1_ref[...])
    h = jnp.maximum(jnp.dot(x, ws1_ref[...],
                            preferred_element_type=jnp.float32)
                    + bs1_ref[...], 0.0)
    sf_ref[...] = (jnp.dot(h, ws2_ref[...],
                           preferred_element_type=jnp.float32) + bs2_ref[...])


def _tc_pre(x, emb, Wt1, bt1, Ws1, bs1, Ws2, bs2):
    return pl.pallas_call(
        _tc_pre_body,
        out_shape=[
            jax.ShapeDtypeStruct((N, HID), jnp.float32),
            jax.ShapeDtypeStruct((NT, HID), jnp.float32),
            jax.ShapeDtypeStruct((N, HALF), jnp.float32),
        ],
    )(x, emb, Wt1, bt1.reshape(1, HID), Ws1, bs1.reshape(1, IN),
      Ws2, bs2.reshape(1, HALF))


def _sc_edges_body(eidx_h, y0_h, y1_h, t0_h, t1_h,
                   h0_out, h1_out, d_out,
                   h_sh, t_v, idx0, idx1, sb0, sb1,
                   rows0, rows1, deg_v,
                   isem0, isem1, gsem0, gsem1, ssem0, ssem1):
    c = lax.axis_index("c")
    s = lax.axis_index("s")

    zero16 = jnp.zeros((16,), jnp.float32)
    one16 = jnp.full((16,), 1.0, jnp.float32)
    lanes = lax.iota(jnp.int32, 16)
    rows = (rows0, rows1)
    idxb = (idx0, idx1)
    sbuf = (sb0, sb1)
    isem = (isem0, isem1)
    gsem = (gsem0, gsem1)
    ssem = (ssem0, ssem1)

    def z_row(i, carry):
        for j in range(8):
            rows0[i, pl.ds(j * 16, 16)] = zero16
        return carry
    lax.fori_loop(0, CH, z_row, 0)

    def zdeg(i, carry):
        deg_v[pl.ds(i * 16, 16)] = zero16
        return carry
    lax.fori_loop(0, 640, zdeg, 0)

    r0 = s * 624
    for k in range(7):
        pltpu.sync_copy(rows0, h_sh.at[pl.ds(r0 + k * CH, CH)])

    @pl.when(s < 15)
    def _():
        pltpu.sync_copy(rows0.at[pl.ds(0, 64)], h_sh.at[pl.ds(r0 + 560, 64)])

    @pl.when(s == 15)
    def _():
        pltpu.sync_copy(rows0, h_sh.at[pl.ds(r0 + 560, CH)])

    @pl.when(c == 0)
    def _():
        pltpu.sync_copy(t0_h, t_v)

    @pl.when(c == 1)
    def _():
        pltpu.sync_copy(t1_h, t_v)

    plsc.subcore_barrier()

    row_t = s * NCHUNK

    def idx_desc(b, il):
        return pltpu.make_async_copy(eidx_h.at[row_t + il], idxb[b], isem[b])

    def gather_desc(b):
        d0 = pltpu.make_async_copy(y0_h.at[idxb[b].at[0]], rows[b], gsem[b])
        d1 = pltpu.make_async_copy(y1_h.at[idxb[b].at[0]], rows[b], gsem[b])
        return d0, d1

    def issue_gather(b):
        d0, d1 = gather_desc(b)

        @pl.when(c == 0)
        def _():
            d0.start()

        @pl.when(c == 1)
        def _():
            d1.start()

    def scat_desc(b):
        return pltpu.make_async_copy(rows[b], h_sh.at[sbuf[b].at[0]], ssem[b])

    def compute(b):
        for r in (0, 1):
            for j in range(5):
                sl = pl.ds(j * 16, 16)
                sbuf[b][r, sl] = idxb[b][r + 1, sl]

        def grp(g, gcarry):
            et16 = sbuf[b][1, pl.ds(g * 16, 16)]

            @pl.when(c == 1)
            def _():
                dst16 = sbuf[b][0, pl.ds(g * 16, 16)]
                for i in range(16):
                    plsc.addupdate_scatter(deg_v, [dst16], one16,
                                           mask=lanes == i)
            for i in range(16):
                te = et16[i]
                e = g * 16 + i
                for j in range(8):
                    sl = pl.ds(j * 16, 16)
                    rows[b][e, sl] = jnp.maximum(
                        rows[b][e, sl] + t_v[te, sl], 0.0)
            return gcarry
        lax.fori_loop(0, CH // 16, grp, 0)

    pltpu.sync_copy(eidx_h.at[row_t], idx0)
    idx_desc(1, 1).start()
    issue_gather(0)

    def pair(g, carry):
        for b in (0, 1):
            il = 2 * g + b
            nb = 1 - b
            if b == 0:
                @pl.when(g > 0)
                def _():
                    scat_desc(1).wait()
                idx_desc(nb, 0).wait()
                issue_gather(nb)
            else:
                scat_desc(0).wait()

                @pl.when(g < NCHUNK // 2 - 1)
                def _():
                    idx_desc(nb, 0).wait()
                    issue_gather(nb)
            gather_desc(b)[0].wait()
            compute(b)

            @pl.when(g < NCHUNK // 2 - 1)
            def _():
                idx_desc(b, il + 2).start()
            pltpu.async_copy(rows[b], h_sh.at[sbuf[b].at[0]], ssem[b],
                             add=True)
        return carry

    lax.fori_loop(0, NCHUNK // 2, pair, 0)
    scat_desc(1).wait()
    plsc.subcore_barrier()

    def stripe_out(h_out):
        for k in range(7):
            pltpu.sync_copy(h_sh.at[pl.ds(r0 + k * CH, CH)], rows0)
            pltpu.sync_copy(rows0, h_out.at[pl.ds(r0 + k * CH, CH)])

        @pl.when(s < 15)
        def _():
            pltpu.sync_copy(h_sh.at[pl.ds(r0 + 560, 64)],
                            rows0.at[pl.ds(0, 64)])
            pltpu.sync_copy(rows0.at[pl.ds(0, 64)],
                            h_out.at[pl.ds(r0 + 560, 64)])

        @pl.when(s == 15)
        def _():
            pltpu.sync_copy(h_sh.at[pl.ds(r0 + 560, CH)], rows0)
            pltpu.sync_copy(rows0, h_out.at[pl.ds(r0 + 560, CH)])

    @pl.when(c == 0)
    def _():
        stripe_out(h0_out)

    @pl.when(c == 1)
    def _():
        stripe_out(h1_out)
        pltpu.sync_copy(deg_v, d_out.at[s])


def _sc_edges(src, dst, et, y0, y1, t0, t1):
    eidx = jnp.stack([src.reshape(E // CH, CH), dst.reshape(E // CH, CH),
                      et.reshape(E // CH, CH)], axis=1)
    mesh = plsc.VectorSubcoreMesh(core_axis_name="c", subcore_axis_name="s")
    f = pl.kernel(
        _sc_edges_body,
        out_type=[
            jax.ShapeDtypeStruct((N, IN), jnp.float32),
            jax.ShapeDtypeStruct((N, IN), jnp.float32),
            jax.ShapeDtypeStruct((NSUB, 10240), jnp.float32),
        ],
        mesh=mesh,
        scratch_types=[
            pltpu.VMEM_SHARED((N, IN), jnp.float32),
            pltpu.VMEM((NT, IN), jnp.float32),
            pltpu.VMEM((3, CH), jnp.int32),
            pltpu.VMEM((3, CH), jnp.int32),
            pltpu.VMEM((2, CH), jnp.int32),
            pltpu.VMEM((2, CH), jnp.int32),
            pltpu.VMEM((CH, IN), jnp.float32),
            pltpu.VMEM((CH, IN), jnp.float32),
            pltpu.VMEM((10240,), jnp.float32),
            pltpu.SemaphoreType.DMA,
            pltpu.SemaphoreType.DMA,
            pltpu.SemaphoreType.DMA,
            pltpu.SemaphoreType.DMA,
            pltpu.SemaphoreType.DMA,
            pltpu.SemaphoreType.DMA,
        ],
        compiler_params=pltpu.CompilerParams(use_tc_tiling_on_sc=False,
                                             needs_layout_passes=False),
    )
    return f(eidx, y0, y1, t0, t1)


def _tc_post_body(h0_ref, h1_ref, d_ref, sf_ref, wt2_ref, bt2_ref, out_ref):
    wt2 = wt2_ref[...]
    bt2 = bt2_ref[...]
    bt2_rep = jnp.broadcast_to(bt2, (NSUB, HALF))
    extra = lax.dot_general(d_ref[...], bt2_rep,
                            (((0,), (0,)), ((), ())),
                            preferred_element_type=jnp.float32)
    msg = (jnp.dot(h0_ref[...], wt2[:IN, :], preferred_element_type=jnp.float32)
           + jnp.dot(h1_ref[...], wt2[IN:, :], preferred_element_type=jnp.float32)
           + extra[:N, :])
    out_ref[...] = jnp.concatenate([sf_ref[...], msg], axis=-1)


def _tc_post(h0, h1, d, sf, Wt2, bt2):
    return pl.pallas_call(
        _tc_post_body,
        out_shape=jax.ShapeDtypeStruct((N, 2 * HALF), jnp.float32),
    )(h0, h1, d, sf, Wt2, bt2.reshape(1, HALF))


def kernel(x, edge_index, edge_type, emb, Ws1, bs1, Ws2, bs2,
           Wt1, bt1, Wt2, bt2):
    src = edge_index[0].astype(jnp.int32)
    dst = edge_index[1].astype(jnp.int32)
    et = edge_type.astype(jnp.int32)

    y, t, sf = _tc_pre(x, emb, Wt1, bt1, Ws1, bs1, Ws2, bs2)
    y0 = y[:, :IN] + 0.0
    y1 = y[:, IN:] + 0.0
    t0 = t[:, :IN] + 0.0
    t1 = t[:, IN:] + 0.0

    h0, h1, d = _sc_edges(src, dst, et, y0, y1, t0, t1)
    return _tc_post(h0, h1, d, sf, Wt2, bt2)


# speedup vs baseline: 6.2227x; 2.4354x over previous
import jax
import jax.numpy as jnp
from jax import lax
from jax.experimental import pallas as pl
from jax.experimental.pallas import tpu as pltpu
from jax.experimental.pallas import tpu_sc as plsc

N = 10000
E = 320000
IN = 128
HID = 256
HALF = 64
NT = 16
NSUB = 16
CH = 80
EPT = E // NSUB
NCHUNK = EPT // CH
NB = 25
BN = N // NB


def _tc_pre_body(x_ref, emb_ref, wt1_ref, bt1_ref, ws1_ref, bs1_ref,
                 ws2_ref, bs2_ref, yt0_ref, yt1_ref, sf_ref):
    x = x_ref[...]
    wt1 = wt1_ref[...]
    y = jnp.dot(x, wt1[IN:, :], preferred_element_type=jnp.float32)
    t = (jnp.dot(emb_ref[...], wt1[:IN, :],
                 preferred_element_type=jnp.float32) + bt1_ref[...])
    y0 = y[:, :IN]
    y1 = y[:, IN:]
    for ti in range(NT):
        yt0_ref[:, ti, :] = jnp.maximum(y0 + t[ti:ti + 1, :IN], 0.0)
        yt1_ref[:, ti, :] = jnp.maximum(y1 + t[ti:ti + 1, IN:], 0.0)
    h = jnp.maximum(jnp.dot(x, ws1_ref[...],
                            preferred_element_type=jnp.float32)
                    + bs1_ref[...], 0.0)
    sf_ref[...] = (jnp.dot(h, ws2_ref[...],
                           preferred_element_type=jnp.float32) + bs2_ref[...])


def _tc_pre(x, emb, Wt1, bt1, Ws1, bs1, Ws2, bs2):
    return pl.pallas_call(
        _tc_pre_body,
        grid=(NB,),
        in_specs=[
            pl.BlockSpec((BN, IN), lambda i: (i, 0)),
            pl.BlockSpec((NT, IN), lambda i: (0, 0)),
            pl.BlockSpec((HID, HID), lambda i: (0, 0)),
            pl.BlockSpec((1, HID), lambda i: (0, 0)),
            pl.BlockSpec((IN, IN), lambda i: (0, 0)),
            pl.BlockSpec((1, IN), lambda i: (0, 0)),
            pl.BlockSpec((IN, HALF), lambda i: (0, 0)),
            pl.BlockSpec((1, HALF), lambda i: (0, 0)),
        ],
        out_specs=[
            pl.BlockSpec((BN, NT, IN), lambda i: (i, 0, 0)),
            pl.BlockSpec((BN, NT, IN), lambda i: (i, 0, 0)),
            pl.BlockSpec((BN, HALF), lambda i: (i, 0)),
        ],
        out_shape=[
            jax.ShapeDtypeStruct((N, NT, IN), jnp.float32),
            jax.ShapeDtypeStruct((N, NT, IN), jnp.float32),
            jax.ShapeDtypeStruct((N, HALF), jnp.float32),
        ],
    )(x, emb, Wt1, bt1.reshape(1, HID), Ws1, bs1.reshape(1, IN),
      Ws2, bs2.reshape(1, HALF))


def _sc_edges_body(eidx_h, yt0_h, yt1_h,
                   h0_out, h1_out, d_out,
                   h_sh, idx0, idx1, sb0, sb1, rows0, rows1, deg_v,
                   isem0, isem1, gsem0, gsem1, ssem0, ssem1):
    c = lax.axis_index("c")
    s = lax.axis_index("s")

    zero16 = jnp.zeros((16,), jnp.float32)
    one16 = jnp.full((16,), 1.0, jnp.float32)
    lanes = lax.iota(jnp.int32, 16)
    rows = (rows0, rows1)
    idxb = (idx0, idx1)
    sbuf = (sb0, sb1)
    isem = (isem0, isem1)
    gsem = (gsem0, gsem1)
    ssem = (ssem0, ssem1)

    def z_row(i, carry):
        for j in range(8):
            rows0[i, pl.ds(j * 16, 16)] = zero16
        return carry
    lax.fori_loop(0, CH, z_row, 0)

    def zdeg(i, carry):
        deg_v[pl.ds(i * 16, 16)] = zero16
        return carry
    lax.fori_loop(0, 640, zdeg, 0)

    r0 = s * 624
    for k in range(7):
        pltpu.sync_copy(rows0, h_sh.at[pl.ds(r0 + k * CH, CH)])

    @pl.when(s < 15)
    def _():
        pltpu.sync_copy(rows0.at[pl.ds(0, 64)], h_sh.at[pl.ds(r0 + 560, 64)])

    @pl.when(s == 15)
    def _():
        pltpu.sync_copy(rows0, h_sh.at[pl.ds(r0 + 560, CH)])

    plsc.subcore_barrier()

    row_t = s * NCHUNK

    def idx_desc(b, il):
        return pltpu.make_async_copy(eidx_h.at[row_t + il], idxb[b], isem[b])

    def gather_desc(b):
        d0 = pltpu.make_async_copy(yt0_h.at[idxb[b].at[0]], rows[b], gsem[b])
        d1 = pltpu.make_async_copy(yt1_h.at[idxb[b].at[0]], rows[b], gsem[b])
        return d0, d1

    def issue_gather(b):
        d0, d1 = gather_desc(b)

        @pl.when(c == 0)
        def _():
            d0.start()

        @pl.when(c == 1)
        def _():
            d1.start()

    def scat_desc(b):
        return pltpu.make_async_copy(rows[b], h_sh.at[sbuf[b].at[0]], ssem[b])

    def stash_and_deg(b, deg_core):
        for j in range(5):
            sl = pl.ds(j * 16, 16)
            sbuf[b][0, sl] = idxb[b][1, sl]

        @pl.when(c == deg_core)
        def _():
            for g in range(5):
                dst16 = sbuf[b][0, pl.ds(g * 16, 16)]
                for i in range(16):
                    plsc.addupdate_scatter(deg_v, [dst16], one16,
                                           mask=lanes == i)

    pltpu.sync_copy(eidx_h.at[row_t], idx0)
    idx_desc(1, 1).start()
    issue_gather(0)

    def pair(g, carry):
        for b in (0, 1):
            il = 2 * g + b
            nb = 1 - b
            if b == 0:
                @pl.when(g > 0)
                def _():
                    scat_desc(1).wait()
                idx_desc(nb, 0).wait()
                issue_gather(nb)
            else:
                scat_desc(0).wait()

                @pl.when(g < NCHUNK // 2 - 1)
                def _():
                    idx_desc(nb, 0).wait()
                    issue_gather(nb)
            gather_desc(b)[0].wait()
            stash_and_deg(b, b)

            @pl.when(g < NCHUNK // 2 - 1)
            def _():
                idx_desc(b, il + 2).start()
            pltpu.async_copy(rows[b], h_sh.at[sbuf[b].at[0]], ssem[b],
                             add=True)
        return carry

    lax.fori_loop(0, NCHUNK // 2, pair, 0)
    scat_desc(1).wait()
    plsc.subcore_barrier()

    def stripe_out(h_out):
        for k in range(7):
            pltpu.sync_copy(h_sh.at[pl.ds(r0 + k * CH, CH)], rows0)
            pltpu.sync_copy(rows0, h_out.at[pl.ds(r0 + k * CH, CH)])

        @pl.when(s < 15)
        def _():
            pltpu.sync_copy(h_sh.at[pl.ds(r0 + 560, 64)],
                            rows0.at[pl.ds(0, 64)])
            pltpu.sync_copy(rows0.at[pl.ds(0, 64)],
                            h_out.at[pl.ds(r0 + 560, 64)])

        @pl.when(s == 15)
        def _():
            pltpu.sync_copy(h_sh.at[pl.ds(r0 + 560, CH)], rows0)
            pltpu.sync_copy(rows0, h_out.at[pl.ds(r0 + 560, CH)])

    @pl.when(c == 0)
    def _():
        stripe_out(h0_out)

    @pl.when(c == 1)
    def _():
        stripe_out(h1_out)

    pltpu.sync_copy(deg_v, d_out.at[c * NSUB + s])


def _sc_edges(cidx, dst, yt0, yt1):
    eidx = jnp.stack([cidx.reshape(E // CH, CH), dst.reshape(E // CH, CH)],
                     axis=1)
    mesh = plsc.VectorSubcoreMesh(core_axis_name="c", subcore_axis_name="s")
    f = pl.kernel(
        _sc_edges_body,
        out_type=[
            jax.ShapeDtypeStruct((N, IN), jnp.float32),
            jax.ShapeDtypeStruct((N, IN), jnp.float32),
            jax.ShapeDtypeStruct((2 * NSUB, 10240), jnp.float32),
        ],
        mesh=mesh,
        scratch_types=[
            pltpu.VMEM_SHARED((N, IN), jnp.float32),
            pltpu.VMEM((2, CH), jnp.int32),
            pltpu.VMEM((2, CH), jnp.int32),
            pltpu.VMEM((1, CH), jnp.int32),
            pltpu.VMEM((1, CH), jnp.int32),
            pltpu.VMEM((CH, IN), jnp.float32),
            pltpu.VMEM((CH, IN), jnp.float32),
            pltpu.VMEM((10240,), jnp.float32),
            pltpu.SemaphoreType.DMA,
            pltpu.SemaphoreType.DMA,
            pltpu.SemaphoreType.DMA,
            pltpu.SemaphoreType.DMA,
            pltpu.SemaphoreType.DMA,
            pltpu.SemaphoreType.DMA,
        ],
        compiler_params=pltpu.CompilerParams(use_tc_tiling_on_sc=False,
                                             needs_layout_passes=False),
    )
    return f(eidx, yt0.reshape(N * NT, IN), yt1.reshape(N * NT, IN))


def _tc_post_body(h0_ref, h1_ref, d_ref, sf_ref, wt2_ref, bt2_ref, out_ref):
    wt2 = wt2_ref[...]
    bt2 = bt2_ref[...]
    bt2_rep = jnp.broadcast_to(bt2, (2 * NSUB, HALF))
    extra = lax.dot_general(d_ref[...], bt2_rep,
                            (((0,), (0,)), ((), ())),
                            preferred_element_type=jnp.float32)
    msg = (jnp.dot(h0_ref[...], wt2[:IN, :], preferred_element_type=jnp.float32)
           + jnp.dot(h1_ref[...], wt2[IN:, :], preferred_element_type=jnp.float32)
           + extra[:N, :])
    out_ref[...] = jnp.concatenate([sf_ref[...], msg], axis=-1)


def _tc_post(h0, h1, d, sf, Wt2, bt2):
    return pl.pallas_call(
        _tc_post_body,
        out_shape=jax.ShapeDtypeStruct((N, 2 * HALF), jnp.float32),
    )(h0, h1, d, sf, Wt2, bt2.reshape(1, HALF))


def kernel(x, edge_index, edge_type, emb, Ws1, bs1, Ws2, bs2,
           Wt1, bt1, Wt2, bt2):
    src = edge_index[0].astype(jnp.int32)
    dst = edge_index[1].astype(jnp.int32)
    et = edge_type.astype(jnp.int32)
    cidx = src * NT + et

    yt0, yt1, sf = _tc_pre(x, emb, Wt1, bt1, Ws1, bs1, Ws2, bs2)
    h0, h1, d = _sc_edges(cidx, dst, yt0, yt1)
    return _tc_post(h0, h1, d, sf, Wt2, bt2)


# speedup vs baseline: 7.7480x; 1.2451x over previous
import jax
import jax.numpy as jnp
from jax import lax
from jax.experimental import pallas as pl
from jax.experimental.pallas import tpu as pltpu
from jax.experimental.pallas import tpu_sc as plsc

N = 10000
E = 320000
IN = 128
HID = 256
HALF = 64
NT = 16
NSUB = 16
CH = 80
EPT = E // NSUB
NCHUNK = EPT // CH
NB = 25
BN = N // NB


def _tc_pre_body(x_ref, emb_ref, wt1_ref, bt1_ref, ws1_ref, bs1_ref,
                 ws2_ref, bs2_ref, yt0_ref, yt1_ref, sf_ref):
    x = x_ref[...]
    wt1 = wt1_ref[...]
    y = jnp.dot(x, wt1[IN:, :], preferred_element_type=jnp.float32)
    t = (jnp.dot(emb_ref[...], wt1[:IN, :],
                 preferred_element_type=jnp.float32) + bt1_ref[...])
    y0 = y[:, :IN]
    y1 = y[:, IN:]
    for ti in range(NT):
        yt0_ref[ti, :, :] = jnp.maximum(y0 + t[ti:ti + 1, :IN], 0.0)
        yt1_ref[ti, :, :] = jnp.maximum(y1 + t[ti:ti + 1, IN:], 0.0)
    h = jnp.maximum(jnp.dot(x, ws1_ref[...],
                            preferred_element_type=jnp.float32)
                    + bs1_ref[...], 0.0)
    sf_ref[...] = (jnp.dot(h, ws2_ref[...],
                           preferred_element_type=jnp.float32) + bs2_ref[...])


def _tc_pre(x, emb, Wt1, bt1, Ws1, bs1, Ws2, bs2):
    return pl.pallas_call(
        _tc_pre_body,
        grid=(NB,),
        in_specs=[
            pl.BlockSpec((BN, IN), lambda i: (i, 0)),
            pl.BlockSpec((NT, IN), lambda i: (0, 0)),
            pl.BlockSpec((HID, HID), lambda i: (0, 0)),
            pl.BlockSpec((1, HID), lambda i: (0, 0)),
            pl.BlockSpec((IN, IN), lambda i: (0, 0)),
            pl.BlockSpec((1, IN), lambda i: (0, 0)),
            pl.BlockSpec((IN, HALF), lambda i: (0, 0)),
            pl.BlockSpec((1, HALF), lambda i: (0, 0)),
        ],
        out_specs=[
            pl.BlockSpec((NT, BN, IN), lambda i: (0, i, 0)),
            pl.BlockSpec((NT, BN, IN), lambda i: (0, i, 0)),
            pl.BlockSpec((BN, HALF), lambda i: (i, 0)),
        ],
        out_shape=[
            jax.ShapeDtypeStruct((NT, N, IN), jnp.float32),
            jax.ShapeDtypeStruct((NT, N, IN), jnp.float32),
            jax.ShapeDtypeStruct((N, HALF), jnp.float32),
        ],
    )(x, emb, Wt1, bt1.reshape(1, HID), Ws1, bs1.reshape(1, IN),
      Ws2, bs2.reshape(1, HALF))


def _sc_edges_body(eidx_h, yt0_h, yt1_h,
                   h0_out, h1_out, d_out,
                   h_sh, idx0, idx1, sb0, sb1, rows0, rows1, deg_v,
                   isem0, isem1, gsem0, gsem1, ssem0, ssem1):
    c = lax.axis_index("c")
    s = lax.axis_index("s")

    zero16 = jnp.zeros((16,), jnp.float32)
    one16 = jnp.full((16,), 1.0, jnp.float32)
    lanes = lax.iota(jnp.int32, 16)
    rows = (rows0, rows1)
    idxb = (idx0, idx1)
    sbuf = (sb0, sb1)
    isem = (isem0, isem1)
    gsem = (gsem0, gsem1)
    ssem = (ssem0, ssem1)

    def z_row(i, carry):
        for j in range(8):
            rows0[i, pl.ds(j * 16, 16)] = zero16
        return carry
    lax.fori_loop(0, CH, z_row, 0)

    def zdeg(i, carry):
        deg_v[pl.ds(i * 16, 16)] = zero16
        return carry
    lax.fori_loop(0, 640, zdeg, 0)

    r0 = s * 624
    for k in range(7):
        pltpu.sync_copy(rows0, h_sh.at[pl.ds(r0 + k * CH, CH)])

    @pl.when(s < 15)
    def _():
        pltpu.sync_copy(rows0.at[pl.ds(0, 64)], h_sh.at[pl.ds(r0 + 560, 64)])

    @pl.when(s == 15)
    def _():
        pltpu.sync_copy(rows0, h_sh.at[pl.ds(r0 + 560, CH)])

    plsc.subcore_barrier()

    row_t = s * NCHUNK

    def idx_desc(b, il):
        return pltpu.make_async_copy(eidx_h.at[row_t + il], idxb[b], isem[b])

    def gather_descs(b, yt_h):
        return [
            pltpu.make_async_copy(yt_h.at[idxb[b].at[0, pl.ds(0, 40)]],
                                  rows[b].at[pl.ds(0, 40)], gsem[b]),
            pltpu.make_async_copy(yt_h.at[idxb[b].at[0, pl.ds(40, 40)]],
                                  rows[b].at[pl.ds(40, 40)], gsem[b]),
        ]

    def issue_gather(b):
        @pl.when(c == 0)
        def _():
            for d in gather_descs(b, yt0_h):
                d.start()

        @pl.when(c == 1)
        def _():
            for d in gather_descs(b, yt1_h):
                d.start()

    def scat_desc(b):
        return pltpu.make_async_copy(rows[b], h_sh.at[sbuf[b].at[0]], ssem[b])

    def stash_and_deg(b, deg_core):
        for j in range(5):
            sl = pl.ds(j * 16, 16)
            sbuf[b][0, sl] = idxb[b][1, sl]

        @pl.when(c == deg_core)
        def _():
            for g in range(5):
                dst16 = sbuf[b][0, pl.ds(g * 16, 16)]
                for i in range(16):
                    plsc.addupdate_scatter(deg_v, [dst16], one16,
                                           mask=lanes == i)

    pltpu.sync_copy(eidx_h.at[row_t], idx0)
    idx_desc(1, 1).start()
    issue_gather(0)

    def pair(g, carry):
        for b in (0, 1):
            il = 2 * g + b
            nb = 1 - b
            if b == 0:
                @pl.when(g > 0)
                def _():
                    scat_desc(1).wait()
                idx_desc(nb, 0).wait()
                issue_gather(nb)
            else:
                scat_desc(0).wait()

                @pl.when(g < NCHUNK // 2 - 1)
                def _():
                    idx_desc(nb, 0).wait()
                    issue_gather(nb)
            for d in gather_descs(b, yt0_h):
                d.wait()
            stash_and_deg(b, b)

            @pl.when(g < NCHUNK // 2 - 1)
            def _():
                idx_desc(b, il + 2).start()
            pltpu.async_copy(rows[b], h_sh.at[sbuf[b].at[0]], ssem[b],
                             add=True)
        return carry

    lax.fori_loop(0, NCHUNK // 2, pair, 0)
    scat_desc(1).wait()
    plsc.subcore_barrier()

    def stripe_out(h_out):
        for k in range(7):
            pltpu.sync_copy(h_sh.at[pl.ds(r0 + k * CH, CH)], rows0)
            pltpu.sync_copy(rows0, h_out.at[pl.ds(r0 + k * CH, CH)])

        @pl.when(s < 15)
        def _():
            pltpu.sync_copy(h_sh.at[pl.ds(r0 + 560, 64)],
                            rows0.at[pl.ds(0, 64)])
            pltpu.sync_copy(rows0.at[pl.ds(0, 64)],
                            h_out.at[pl.ds(r0 + 560, 64)])

        @pl.when(s == 15)
        def _():
            pltpu.sync_copy(h_sh.at[pl.ds(r0 + 560, CH)], rows0)
            pltpu.sync_copy(rows0, h_out.at[pl.ds(r0 + 560, CH)])

    @pl.when(c == 0)
    def _():
        stripe_out(h0_out)

    @pl.when(c == 1)
    def _():
        stripe_out(h1_out)

    pltpu.sync_copy(deg_v, d_out.at[c * NSUB + s])


def _sc_edges(cidx, dst, yt0, yt1):
    eidx = jnp.stack([cidx.reshape(E // CH, CH), dst.reshape(E // CH, CH)],
                     axis=1)
    mesh = plsc.VectorSubcoreMesh(core_axis_name="c", subcore_axis_name="s")
    f = pl.kernel(
        _sc_edges_body,
        out_type=[
            jax.ShapeDtypeStruct((N, IN), jnp.float32),
            jax.ShapeDtypeStruct((N, IN), jnp.float32),
            jax.ShapeDtypeStruct((2 * NSUB, 10240), jnp.float32),
        ],
        mesh=mesh,
        scratch_types=[
            pltpu.VMEM_SHARED((N, IN), jnp.float32),
            pltpu.VMEM((2, CH), jnp.int32),
            pltpu.VMEM((2, CH), jnp.int32),
            pltpu.VMEM((1, CH), jnp.int32),
            pltpu.VMEM((1, CH), jnp.int32),
            pltpu.VMEM((CH, IN), jnp.float32),
            pltpu.VMEM((CH, IN), jnp.float32),
            pltpu.VMEM((10240,), jnp.float32),
            pltpu.SemaphoreType.DMA,
            pltpu.SemaphoreType.DMA,
            pltpu.SemaphoreType.DMA,
            pltpu.SemaphoreType.DMA,
            pltpu.SemaphoreType.DMA,
            pltpu.SemaphoreType.DMA,
        ],
        compiler_params=pltpu.CompilerParams(use_tc_tiling_on_sc=False,
                                             needs_layout_passes=False),
    )
    return f(eidx, yt0.reshape(N * NT, IN), yt1.reshape(N * NT, IN))


def _tc_post_body(h0_ref, h1_ref, d_ref, sf_ref, wt2_ref, bt2_ref, out_ref):
    wt2 = wt2_ref[...]
    bt2 = bt2_ref[...]
    bt2_rep = jnp.broadcast_to(bt2, (2 * NSUB, HALF))
    extra = lax.dot_general(d_ref[...], bt2_rep,
                            (((0,), (0,)), ((), ())),
                            preferred_element_type=jnp.float32)
    msg = (jnp.dot(h0_ref[...], wt2[:IN, :], preferred_element_type=jnp.float32)
           + jnp.dot(h1_ref[...], wt2[IN:, :], preferred_element_type=jnp.float32)
           + extra[:N, :])
    out_ref[...] = jnp.concatenate([sf_ref[...], msg], axis=-1)


def _tc_post(h0, h1, d, sf, Wt2, bt2):
    return pl.pallas_call(
        _tc_post_body,
        out_shape=jax.ShapeDtypeStruct((N, 2 * HALF), jnp.float32),
    )(h0, h1, d, sf, Wt2, bt2.reshape(1, HALF))


def kernel(x, edge_index, edge_type, emb, Ws1, bs1, Ws2, bs2,
           Wt1, bt1, Wt2, bt2):
    src = edge_index[0].astype(jnp.int32)
    dst = edge_index[1].astype(jnp.int32)
    et = edge_type.astype(jnp.int32)
    cidx = et * N + src

    yt0, yt1, sf = _tc_pre(x, emb, Wt1, bt1, Ws1, bs1, Ws2, bs2)
    h0, h1, d = _sc_edges(cidx, dst, yt0, yt1)
    return _tc_post(h0, h1, d, sf, Wt2, bt2)


# speedup vs baseline: 10.2205x; 1.3191x over previous
import jax
import jax.numpy as jnp
from jax import lax
from jax.experimental import pallas as pl
from jax.experimental.pallas import tpu as pltpu
from jax.experimental.pallas import tpu_sc as plsc

N = 10000
E = 320000
IN = 128
HID = 256
HALF = 64
NT = 16
NSUB = 16
CH = 80
CPT = E // (2 * NSUB * CH)
NB = 25
BN = N // NB


def _tc_pre_body(x_ref, emb_ref, wt1_ref, bt1_ref, wt2_ref, bt2_ref,
                 ws1_ref, bs1_ref, ws2_ref, bs2_ref, z_ref, sf_ref):
    x = x_ref[...]
    wt1 = wt1_ref[...]
    wt2 = wt2_ref[...]
    bt2 = bt2_ref[...]
    y = jnp.dot(x, wt1[IN:, :], preferred_element_type=jnp.float32)
    t = (jnp.dot(emb_ref[...], wt1[:IN, :],
                 preferred_element_type=jnp.float32) + bt1_ref[...])
    for ti in range(NT):
        r = jnp.maximum(y + t[ti:ti + 1, :], 0.0)
        z_ref[ti, :, :] = (jnp.dot(r, wt2,
                                   preferred_element_type=jnp.float32) + bt2)
    h = jnp.maximum(jnp.dot(x, ws1_ref[...],
                            preferred_element_type=jnp.float32)
                    + bs1_ref[...], 0.0)
    sf_ref[...] = (jnp.dot(h, ws2_ref[...],
                           preferred_element_type=jnp.float32) + bs2_ref[...])


def _tc_pre(x, emb, Wt1, bt1, Wt2, bt2, Ws1, bs1, Ws2, bs2):
    return pl.pallas_call(
        _tc_pre_body,
        grid=(NB,),
        in_specs=[
            pl.BlockSpec((BN, IN), lambda i: (i, 0)),
            pl.BlockSpec((NT, IN), lambda i: (0, 0)),
            pl.BlockSpec((HID, HID), lambda i: (0, 0)),
            pl.BlockSpec((1, HID), lambda i: (0, 0)),
            pl.BlockSpec((HID, HALF), lambda i: (0, 0)),
            pl.BlockSpec((1, HALF), lambda i: (0, 0)),
            pl.BlockSpec((IN, IN), lambda i: (0, 0)),
            pl.BlockSpec((1, IN), lambda i: (0, 0)),
            pl.BlockSpec((IN, HALF), lambda i: (0, 0)),
            pl.BlockSpec((1, HALF), lambda i: (0, 0)),
        ],
        out_specs=[
            pl.BlockSpec((NT, BN, HALF), lambda i: (0, i, 0)),
            pl.BlockSpec((BN, HALF), lambda i: (i, 0)),
        ],
        out_shape=[
            jax.ShapeDtypeStruct((NT, N, HALF), jnp.float32),
            jax.ShapeDtypeStruct((N, HALF), jnp.float32),
        ],
    )(x, emb, Wt1, bt1.reshape(1, HID), Wt2, bt2.reshape(1, HALF),
      Ws1, bs1.reshape(1, IN), Ws2, bs2.reshape(1, HALF))


def _sc_edges_body(eidx_h, z_h,
                   h0_out, h1_out,
                   h_sh, idx0, idx1, sb0, sb1, rows0, rows1,
                   isem0, isem1, gsem0, gsem1, ssem0, ssem1):
    c = lax.axis_index("c")
    s = lax.axis_index("s")

    zero16 = jnp.zeros((16,), jnp.float32)
    rows = (rows0, rows1)
    idxb = (idx0, idx1)
    sbuf = (sb0, sb1)
    isem = (isem0, isem1)
    gsem = (gsem0, gsem1)
    ssem = (ssem0, ssem1)

    def z_row(i, carry):
        for j in range(4):
            rows0[i, pl.ds(j * 16, 16)] = zero16
        return carry
    lax.fori_loop(0, CH, z_row, 0)

    r0 = s * 624
    for k in range(7):
        pltpu.sync_copy(rows0, h_sh.at[pl.ds(r0 + k * CH, CH)])

    @pl.when(s < 15)
    def _():
        pltpu.sync_copy(rows0.at[pl.ds(0, 64)], h_sh.at[pl.ds(r0 + 560, 64)])

    @pl.when(s == 15)
    def _():
        pltpu.sync_copy(rows0, h_sh.at[pl.ds(r0 + 560, CH)])

    plsc.subcore_barrier()

    row_t = c * (NSUB * CPT) + s * CPT

    def idx_desc(b, il):
        return pltpu.make_async_copy(eidx_h.at[row_t + il], idxb[b], isem[b])

    def gather_desc(b):
        return pltpu.make_async_copy(z_h.at[idxb[b].at[0]], rows[b], gsem[b])

    def scat_desc(b):
        return pltpu.make_async_copy(rows[b], h_sh.at[sbuf[b].at[0]], ssem[b])

    def stash(b):
        for j in range(5):
            sl = pl.ds(j * 16, 16)
            sbuf[b][0, sl] = idxb[b][1, sl]

    def scatter(b):
        pltpu.async_copy(rows[b], h_sh.at[sbuf[b].at[0]], ssem[b], add=True)

    pltpu.sync_copy(eidx_h.at[row_t], idx0)
    idx_desc(1, 1).start()
    gather_desc(0).start()

    def pair(g, carry):
        @pl.when(g > 0)
        def _():
            scat_desc(1).wait()
        idx_desc(1, 0).wait()
        gather_desc(1).start()
        gather_desc(0).wait()
        stash(0)
        idx_desc(0, 2 * g + 2).start()
        scatter(0)

        scat_desc(0).wait()
        idx_desc(0, 0).wait()
        gather_desc(0).start()
        gather_desc(1).wait()
        stash(1)

        @pl.when(g < CPT // 2 - 1)
        def _():
            idx_desc(1, 2 * g + 3).start()
        scatter(1)
        return carry

    lax.fori_loop(0, CPT // 2, pair, 0)

    scat_desc(1).wait()
    gather_desc(0).wait()
    stash(0)
    scatter(0)
    scat_desc(0).wait()
    plsc.subcore_barrier()

    def stripe_out(h_out):
        for k in range(7):
            pltpu.sync_copy(h_sh.at[pl.ds(r0 + k * CH, CH)], rows0)
            pltpu.sync_copy(rows0, h_out.at[pl.ds(r0 + k * CH, CH)])

        @pl.when(s < 15)
        def _():
            pltpu.sync_copy(h_sh.at[pl.ds(r0 + 560, 64)],
                            rows0.at[pl.ds(0, 64)])
            pltpu.sync_copy(rows0.at[pl.ds(0, 64)],
                            h_out.at[pl.ds(r0 + 560, 64)])

        @pl.when(s == 15)
        def _():
            pltpu.sync_copy(h_sh.at[pl.ds(r0 + 560, CH)], rows0)
            pltpu.sync_copy(rows0, h_out.at[pl.ds(r0 + 560, CH)])

    @pl.when(c == 0)
    def _():
        stripe_out(h0_out)

    @pl.when(c == 1)
    def _():
        stripe_out(h1_out)


def _sc_edges(cidx, dst, z):
    eidx = jnp.stack([cidx.reshape(E // CH, CH), dst.reshape(E // CH, CH)],
                     axis=1)
    mesh = plsc.VectorSubcoreMesh(core_axis_name="c", subcore_axis_name="s")
    f = pl.kernel(
        _sc_edges_body,
        out_type=[
            jax.ShapeDtypeStruct((N, HALF), jnp.float32),
            jax.ShapeDtypeStruct((N, HALF), jnp.float32),
        ],
        mesh=mesh,
        scratch_types=[
            pltpu.VMEM_SHARED((N, HALF), jnp.float32),
            pltpu.VMEM((2, CH), jnp.int32),
            pltpu.VMEM((2, CH), jnp.int32),
            pltpu.VMEM((1, CH), jnp.int32),
            pltpu.VMEM((1, CH), jnp.int32),
            pltpu.VMEM((CH, HALF), jnp.float32),
            pltpu.VMEM((CH, HALF), jnp.float32),
            pltpu.SemaphoreType.DMA,
            pltpu.SemaphoreType.DMA,
            pltpu.SemaphoreType.DMA,
            pltpu.SemaphoreType.DMA,
            pltpu.SemaphoreType.DMA,
            pltpu.SemaphoreType.DMA,
        ],
        compiler_params=pltpu.CompilerParams(use_tc_tiling_on_sc=False,
                                             needs_layout_passes=False),
    )
    return f(eidx, z.reshape(NT * N, HALF))


def _tc_post_body(h0_ref, h1_ref, sf_ref, out_ref):
    msg = h0_ref[...] + h1_ref[...]
    out_ref[...] = jnp.concatenate([sf_ref[...], msg], axis=-1)


def _tc_post(h0, h1, sf):
    return pl.pallas_call(
        _tc_post_body,
        out_shape=jax.ShapeDtypeStruct((N, 2 * HALF), jnp.float32),
    )(h0, h1, sf)


def kernel(x, edge_index, edge_type, emb, Ws1, bs1, Ws2, bs2,
           Wt1, bt1, Wt2, bt2):
    src = edge_index[0].astype(jnp.int32)
    dst = edge_index[1].astype(jnp.int32)
    et = edge_type.astype(jnp.int32)
    cidx = et * N + src

    z, sf = _tc_pre(x, emb, Wt1, bt1, Wt2, bt2, Ws1, bs1, Ws2, bs2)
    h0, h1 = _sc_edges(cidx, dst, z)
    return _tc_post(h0, h1, sf)
